# baseline (speedup 1.0000x reference)
import functools

import jax
import jax.numpy as jnp
from jax import lax
from jax.experimental import pallas as pl
from jax.experimental.pallas import tpu as pltpu
from jax.experimental.pallas import tpu_sc as plsc

N, E, D, P = 10000, 320000, 128, 12

NC, NS, LANES = 2, 16, 16
NW = NC * NS
CHUNK = 128
CPT = 80
E_PAD = NW * CPT * CHUNK
ROWS_PER_SUB = 640
N_PAD = NS * ROWS_PER_SUB
DUMMY_ROW = N_PAD - 8
CW = 128

_mesh = plsc.VectorSubcoreMesh(core_axis_name="c", subcore_axis_name="s")


def _make_sc_aggregate():
    out_types = jax.ShapeDtypeStruct((NC, N_PAD, D), jnp.float32)
    scratch = [pltpu.VMEM((CPT * CHUNK,), jnp.int32),
               pltpu.VMEM((CHUNK,), jnp.int32),
               pltpu.VMEM((CHUNK,), jnp.int32),
               pltpu.VMEM((CHUNK, D), jnp.float32),
               pltpu.VMEM((CHUNK, D), jnp.float32),
               pltpu.VMEM_SHARED((N_PAD, D), jnp.float32),
               pltpu.SemaphoreType.DMA,
               pltpu.SemaphoreType.DMA,
               pltpu.SemaphoreType.DMA,
               pltpu.SemaphoreType.DMA]

    @functools.partial(pl.kernel, out_type=out_types, mesh=_mesh,
                       scratch_types=scratch)
    def sc_kernel(h_hbm, src_hbm, dst_hbm, z_hbm, pp, srcall, dst0, dst1,
                  rows0, rows1, acc, sg0, sg1, sd0, sd1):
        cid = lax.axis_index("c")
        sid = lax.axis_index("s")
        wid = sid * NC + cid
        base = sid * ROWS_PER_SUB
        ebase = wid * CPT * CHUNK

        dstb = (dst0, dst1)
        rows = (rows0, rows1)
        sg = (sg0, sg1)
        sd = (sd0, sd1)

        pltpu.sync_copy(src_hbm.at[pl.ds(ebase, CPT * CHUNK)], srcall)
        pltpu.sync_copy(z_hbm, acc.at[pl.ds(base, ROWS_PER_SUB)])

        def issue(i, b):
            pltpu.async_copy(dst_hbm.at[pl.ds(ebase + i * CHUNK, CHUNK)],
                             dstb[b], sd[b])
            pltpu.async_copy(h_hbm.at[srcall.at[pl.ds(i * CHUNK, CHUNK)]],
                             rows[b], sg[b])

        issue(0, 0)
        issue(1, 1)
        plsc.subcore_barrier()

        @pl.loop(0, CPT // 2)
        def _(j):
            i0 = j * 2
            for b in range(2):
                i = i0 + b
                pltpu.make_async_copy(dst_hbm.at[pl.ds(0, CHUNK)],
                                      dstb[b], sd[b]).wait()
                pltpu.make_async_copy(h_hbm.at[pl.ds(0, CHUNK)],
                                      rows[b], sg[b]).wait()
                pltpu.sync_copy(rows[b], acc.at[dstb[b]], add=True)

                @pl.when(i + 2 < CPT)
                def _():
                    issue(i + 2, b)

        plsc.subcore_barrier()
        pltpu.sync_copy(acc.at[pl.ds(base, ROWS_PER_SUB)],
                        pp.at[cid, pl.ds(base, ROWS_PER_SUB)])

    return sc_kernel


def _make_sc_count():
    out_types = jax.ShapeDtypeStruct((NC, N_PAD, CW), jnp.float32)
    scratch = [pltpu.VMEM((CHUNK,), jnp.int32),
               pltpu.VMEM((CHUNK,), jnp.int32),
               pltpu.VMEM((CHUNK, CW), jnp.float32),
               pltpu.VMEM_SHARED((N_PAD, CW), jnp.float32),
               pltpu.SemaphoreType.DMA,
               pltpu.SemaphoreType.DMA]

    @functools.partial(pl.kernel, out_type=out_types, mesh=_mesh,
                       scratch_types=scratch)
    def sc_kernel(dst_hbm, ones_hbm, zc_hbm, cc, dst0, dst1, ones, cnt,
                  sd0, sd1):
        cid = lax.axis_index("c")
        sid = lax.axis_index("s")
        wid = sid * NC + cid
        base = sid * ROWS_PER_SUB
        ebase = wid * CPT * CHUNK

        dstb = (dst0, dst1)
        sd = (sd0, sd1)

        pltpu.sync_copy(ones_hbm, ones)
        pltpu.sync_copy(zc_hbm, cnt.at[pl.ds(base, ROWS_PER_SUB)])

        def issue(i, b):
            pltpu.async_copy(dst_hbm.at[pl.ds(ebase + i * CHUNK, CHUNK)],
                             dstb[b], sd[b])

        issue(0, 0)
        issue(1, 1)
        plsc.subcore_barrier()

        @pl.loop(0, CPT // 2)
        def _(j):
            i0 = j * 2
            for b in range(2):
                i = i0 + b
                pltpu.make_async_copy(dst_hbm.at[pl.ds(0, CHUNK)],
                                      dstb[b], sd[b]).wait()
                pltpu.sync_copy(ones, cnt.at[dstb[b]], add=True)

                @pl.when(i + 2 < CPT)
                def _():
                    issue(i + 2, b)

        plsc.subcore_barrier()
        pltpu.sync_copy(cnt.at[pl.ds(base, ROWS_PER_SUB)],
                        cc.at[cid, pl.ds(base, ROWS_PER_SUB)])

    return sc_kernel


_sc_aggregate = _make_sc_aggregate()
_sc_count = _make_sc_count()

BM = 1000


def _dot(a, w):
    return lax.dot_general(a, w, (((1,), (1,)), ((), ())),
                           precision=lax.Precision.DEFAULT)


def _row_spec(bm, d):
    return pl.BlockSpec((bm, d), lambda i: (i, 0))


def _full_spec(shape):
    return pl.BlockSpec(shape, lambda i: tuple(0 for _ in shape))


def _tc_matmul_body(h, wr, b, o):
    o[...] = _dot(h[...], wr[...]) + b[...]


def _tc_matmul(h, wr, b):
    return pl.pallas_call(
        _tc_matmul_body,
        grid=(N // BM,),
        in_specs=[_row_spec(BM, D), _full_spec((D, D)), _full_spec((1, D))],
        out_specs=_row_spec(BM, D),
        out_shape=jax.ShapeDtypeStruct((N, D), jnp.float32),
    )(h, wr, b.reshape(1, D))


def _tc_combine_body(p0, p1, c0, c1, m2, wl, o):
    invc = 1.0 / jnp.maximum(c0[:, 0:1] + c1[:, 0:1], 1.0)
    hp = invc * _dot(p0[...] + p1[...], wl[...])
    o[...] = jnp.maximum(hp + m2[...], 0.0)


def _tc_combine(p0, p1, c0, c1, m2, wl):
    return pl.pallas_call(
        _tc_combine_body,
        grid=(N // BM,),
        in_specs=[_row_spec(BM, D), _row_spec(BM, D),
                  _row_spec(BM, CW), _row_spec(BM, CW),
                  _row_spec(BM, D), _full_spec((D, D))],
        out_specs=_row_spec(BM, D),
        out_shape=jax.ShapeDtypeStruct((N, D), jnp.float32),
    )(p0, p1, c0, c1, m2, wl)


def _tc_combine2_body(p0, p1, c0, c1, m2, wl, wout, bout, o):
    invc = 1.0 / jnp.maximum(c0[:, 0:1] + c1[:, 0:1], 1.0)
    h2 = jnp.maximum(invc * _dot(p0[...] + p1[...], wl[...]) + m2[...], 0.0)
    o[...] = _dot(h2, wout[...]) + bout[...]


def _tc_combine2(p0, p1, c0, c1, m2, wl, wout, bout):
    return pl.pallas_call(
        _tc_combine2_body,
        grid=(N // BM,),
        in_specs=[_row_spec(BM, D), _row_spec(BM, D),
                  _row_spec(BM, CW), _row_spec(BM, CW),
                  _row_spec(BM, D), _full_spec((D, D)),
                  _full_spec((P, D)), _full_spec((1, P))],
        out_specs=_row_spec(BM, P),
        out_shape=jax.ShapeDtypeStruct((N, P), jnp.float32),
    )(p0, p1, c0, c1, m2, wl, wout, bout.reshape(1, P))


def kernel(x, edge_index, Wl1, Wr1, b1, Wl2, Wr2, b2, Wout, bout):
    src = edge_index[0]
    dst = edge_index[1]
    pad = E_PAD - E
    ar = jnp.arange(pad, dtype=jnp.int32)
    src_p = jnp.concatenate([src, ar % N])
    dst_p = jnp.concatenate([dst, N + 8 + (ar % (N_PAD - N - 16))])

    zrows = jnp.zeros((ROWS_PER_SUB, D), jnp.float32)
    ones_cw = jnp.ones((CHUNK, CW), jnp.float32)
    zc_cw = jnp.zeros((ROWS_PER_SUB, CW), jnp.float32)
    m2 = _tc_matmul(x, Wr1, b1)
    cc = _sc_count(dst_p, ones_cw, zc_cw)
    pp = _sc_aggregate(x, src_p, dst_p, zrows)
    h1 = _tc_combine(pp[0], pp[1], cc[0], cc[1], m2, Wl1)
    m2b = _tc_matmul(h1, Wr2, b2)
    qq = _sc_aggregate(h1, src_p, dst_p, zrows)
    out = _tc_combine2(qq[0], qq[1], cc[0], cc[1], m2b, Wl2, Wout, bout)
    return out


# speedup vs baseline: 1.0730x; 1.0730x over previous
import functools

import jax
import jax.numpy as jnp
from jax import lax
from jax.experimental import pallas as pl
from jax.experimental.pallas import tpu as pltpu
from jax.experimental.pallas import tpu_sc as plsc

N, E, D, P = 10000, 320000, 128, 12

NC, NS, LANES = 2, 16, 16
NW = NC * NS
CHUNK = 128
CPT = 80
E_PAD = NW * CPT * CHUNK
ROWS_PER_SUB = 640
N_PAD = NS * ROWS_PER_SUB
DUMMY_ROW = N_PAD - 8
CW = 128

_mesh = plsc.VectorSubcoreMesh(core_axis_name="c", subcore_axis_name="s")


NBUF = 4
GCH = 64
GPT = (CPT * CHUNK) // GCH


def _make_sc_aggregate():
    out_types = jax.ShapeDtypeStruct((NC, N_PAD, D), jnp.float32)
    scratch = ([pltpu.VMEM((CPT * CHUNK,), jnp.int32)]
               + [pltpu.VMEM((GCH,), jnp.int32) for _ in range(NBUF)]
               + [pltpu.VMEM((GCH, D), jnp.float32) for _ in range(NBUF)]
               + [pltpu.VMEM_SHARED((N_PAD, D), jnp.float32)]
               + [pltpu.SemaphoreType.DMA for _ in range(2 * NBUF)])

    @functools.partial(pl.kernel, out_type=out_types, mesh=_mesh,
                       scratch_types=scratch)
    def sc_kernel(h_hbm, src_hbm, dst_hbm, z_hbm, pp, srcall, *rest):
        dstb = rest[0:NBUF]
        rows = rest[NBUF:2 * NBUF]
        acc = rest[2 * NBUF]
        sg = rest[2 * NBUF + 1:2 * NBUF + 1 + NBUF]
        sd = rest[2 * NBUF + 1 + NBUF:]

        cid = lax.axis_index("c")
        sid = lax.axis_index("s")
        wid = sid * NC + cid
        base = sid * ROWS_PER_SUB
        ebase = wid * CPT * CHUNK

        pltpu.sync_copy(src_hbm.at[pl.ds(ebase, CPT * CHUNK)], srcall)
        pltpu.sync_copy(z_hbm, acc.at[pl.ds(base, ROWS_PER_SUB)])

        def issue(i, b):
            pltpu.async_copy(dst_hbm.at[pl.ds(ebase + i * GCH, GCH)],
                             dstb[b], sd[b])
            pltpu.async_copy(h_hbm.at[srcall.at[pl.ds(i * GCH, GCH)]],
                             rows[b], sg[b])

        for b in range(NBUF):
            issue(b, b)
        plsc.subcore_barrier()

        @pl.loop(0, GPT // NBUF)
        def _(j):
            i0 = j * NBUF
            for b in range(NBUF):
                i = i0 + b
                pltpu.make_async_copy(dst_hbm.at[pl.ds(0, GCH)],
                                      dstb[b], sd[b]).wait()
                pltpu.make_async_copy(h_hbm.at[pl.ds(0, GCH)],
                                      rows[b], sg[b]).wait()
                pltpu.sync_copy(rows[b], acc.at[dstb[b]], add=True)

                @pl.when(i + NBUF < GPT)
                def _():
                    issue(i + NBUF, b)

        plsc.subcore_barrier()
        pltpu.sync_copy(acc.at[pl.ds(base, ROWS_PER_SUB)],
                        pp.at[cid, pl.ds(base, ROWS_PER_SUB)])

    return sc_kernel


def _make_sc_count():
    out_types = jax.ShapeDtypeStruct((NC, N_PAD, CW), jnp.float32)
    scratch = [pltpu.VMEM((CHUNK,), jnp.int32),
               pltpu.VMEM((CHUNK,), jnp.int32),
               pltpu.VMEM((CHUNK, CW), jnp.float32),
               pltpu.VMEM_SHARED((N_PAD, CW), jnp.float32),
               pltpu.SemaphoreType.DMA,
               pltpu.SemaphoreType.DMA]

    @functools.partial(pl.kernel, out_type=out_types, mesh=_mesh,
                       scratch_types=scratch)
    def sc_kernel(dst_hbm, ones_hbm, zc_hbm, cc, dst0, dst1, ones, cnt,
                  sd0, sd1):
        cid = lax.axis_index("c")
        sid = lax.axis_index("s")
        wid = sid * NC + cid
        base = sid * ROWS_PER_SUB
        ebase = wid * CPT * CHUNK

        dstb = (dst0, dst1)
        sd = (sd0, sd1)

        pltpu.sync_copy(ones_hbm, ones)
        pltpu.sync_copy(zc_hbm, cnt.at[pl.ds(base, ROWS_PER_SUB)])

        def issue(i, b):
            pltpu.async_copy(dst_hbm.at[pl.ds(ebase + i * CHUNK, CHUNK)],
                             dstb[b], sd[b])

        issue(0, 0)
        issue(1, 1)
        plsc.subcore_barrier()

        @pl.loop(0, CPT // 2)
        def _(j):
            i0 = j * 2
            for b in range(2):
                i = i0 + b
                pltpu.make_async_copy(dst_hbm.at[pl.ds(0, CHUNK)],
                                      dstb[b], sd[b]).wait()
                pltpu.sync_copy(ones, cnt.at[dstb[b]], add=True)

                @pl.when(i + 2 < CPT)
                def _():
                    issue(i + 2, b)

        plsc.subcore_barrier()
        pltpu.sync_copy(cnt.at[pl.ds(base, ROWS_PER_SUB)],
                        cc.at[cid, pl.ds(base, ROWS_PER_SUB)])

    return sc_kernel


_sc_aggregate = _make_sc_aggregate()
_sc_count = _make_sc_count()

BM = 1000


def _dot(a, w):
    return lax.dot_general(a, w, (((1,), (1,)), ((), ())),
                           precision=lax.Precision.DEFAULT)


def _row_spec(bm, d):
    return pl.BlockSpec((bm, d), lambda i: (i, 0))


def _full_spec(shape):
    return pl.BlockSpec(shape, lambda i: tuple(0 for _ in shape))


def _tc_matmul_body(h, wr, b, o):
    o[...] = _dot(h[...], wr[...]) + b[...]


def _tc_matmul(h, wr, b):
    return pl.pallas_call(
        _tc_matmul_body,
        grid=(N // BM,),
        in_specs=[_row_spec(BM, D), _full_spec((D, D)), _full_spec((1, D))],
        out_specs=_row_spec(BM, D),
        out_shape=jax.ShapeDtypeStruct((N, D), jnp.float32),
    )(h, wr, b.reshape(1, D))


def _tc_combine_body(p0, p1, c0, c1, m2, wl, o):
    invc = 1.0 / jnp.maximum(c0[:, 0:1] + c1[:, 0:1], 1.0)
    hp = invc * _dot(p0[...] + p1[...], wl[...])
    o[...] = jnp.maximum(hp + m2[...], 0.0)


def _tc_combine(p0, p1, c0, c1, m2, wl):
    return pl.pallas_call(
        _tc_combine_body,
        grid=(N // BM,),
        in_specs=[_row_spec(BM, D), _row_spec(BM, D),
                  _row_spec(BM, CW), _row_spec(BM, CW),
                  _row_spec(BM, D), _full_spec((D, D))],
        out_specs=_row_spec(BM, D),
        out_shape=jax.ShapeDtypeStruct((N, D), jnp.float32),
    )(p0, p1, c0, c1, m2, wl)


def _tc_combine2_body(p0, p1, c0, c1, m2, wl, wout, bout, o):
    invc = 1.0 / jnp.maximum(c0[:, 0:1] + c1[:, 0:1], 1.0)
    h2 = jnp.maximum(invc * _dot(p0[...] + p1[...], wl[...]) + m2[...], 0.0)
    o[...] = _dot(h2, wout[...]) + bout[...]


def _tc_combine2(p0, p1, c0, c1, m2, wl, wout, bout):
    return pl.pallas_call(
        _tc_combine2_body,
        grid=(N // BM,),
        in_specs=[_row_spec(BM, D), _row_spec(BM, D),
                  _row_spec(BM, CW), _row_spec(BM, CW),
                  _row_spec(BM, D), _full_spec((D, D)),
                  _full_spec((P, D)), _full_spec((1, P))],
        out_specs=_row_spec(BM, P),
        out_shape=jax.ShapeDtypeStruct((N, P), jnp.float32),
    )(p0, p1, c0, c1, m2, wl, wout, bout.reshape(1, P))


def kernel(x, edge_index, Wl1, Wr1, b1, Wl2, Wr2, b2, Wout, bout):
    src = edge_index[0]
    dst = edge_index[1]
    pad = E_PAD - E
    ar = jnp.arange(pad, dtype=jnp.int32)
    src_p = jnp.concatenate([src, ar % N])
    dst_p = jnp.concatenate([dst, N + 8 + (ar % (N_PAD - N - 16))])

    zrows = jnp.zeros((ROWS_PER_SUB, D), jnp.float32)
    ones_cw = jnp.ones((CHUNK, CW), jnp.float32)
    zc_cw = jnp.zeros((ROWS_PER_SUB, CW), jnp.float32)
    m2 = _tc_matmul(x, Wr1, b1)
    cc = _sc_count(dst_p, ones_cw, zc_cw)
    pp = _sc_aggregate(x, src_p, dst_p, zrows)
    h1 = _tc_combine(pp[0], pp[1], cc[0], cc[1], m2, Wl1)
    m2b = _tc_matmul(h1, Wr2, b2)
    qq = _sc_aggregate(h1, src_p, dst_p, zrows)
    out = _tc_combine2(qq[0], qq[1], cc[0], cc[1], m2b, Wl2, Wout, bout)
    return out
